# baseline (speedup 1.0000x reference)
import functools

import jax
import jax.numpy as jnp
from jax import lax
from jax.experimental import pallas as pl
from jax.experimental.pallas import tpu as pltpu
from jax.experimental.pallas import tpu_sc as plsc

_N = 10000
_NPAD = 10240
_E = 320000
_D_IN = 128
_D_HID = 128
_D_OUT = 64

_NC = 2
_NS = 16
_NW = _NC * _NS
_CK = 128
_NCH = 80
_EPAD = _NW * _NCH * _CK
_SCK = 64
_GCH = _EPAD // _SCK
_T0 = 176
_T1 = 144
_STG = 16
_NBUF = 4
_RPT = _NPAD // _NS
_DEGW = 128

_mesh = plsc.VectorSubcoreMesh(
    core_axis_name="c", subcore_axis_name="s",
    num_cores=_NC, num_subcores=_NS)


@functools.partial(
    pl.kernel,
    out_type=jax.ShapeDtypeStruct((_NC, _NPAD, _DEGW), jnp.float32),
    mesh=_mesh,
    scratch_types=[
        pltpu.VMEM((_NCH, _CK), jnp.int32),
        pltpu.VMEM((_CK, _DEGW), jnp.float32),
        pltpu.VMEM((16, _DEGW), jnp.float32),
        pltpu.VMEM_SHARED((_NPAD, _DEGW), jnp.float32),
        pltpu.SemaphoreType.DMA,
    ],
)
def _deg_kernel(dst_hbm, out_hbm, dst_v, ones_v, zeros_v, acc_sh, dsem):
    c = lax.axis_index("c")
    s = lax.axis_index("s")
    tid = c * _NS + s
    pltpu.sync_copy(dst_hbm.at[tid], dst_v)
    for i in range(_CK):
        for w in range(_DEGW // 16):
            ones_v[i, 16 * w:16 * (w + 1)] = jnp.ones((16,), jnp.float32)
    for i in range(16):
        for w in range(_DEGW // 16):
            zeros_v[i, 16 * w:16 * (w + 1)] = jnp.zeros((16,), jnp.float32)

    def zero_body(r, carry):
        pltpu.async_copy(zeros_v, acc_sh.at[pl.ds(s * _RPT + r * 16, 16)], dsem)
        return carry
    lax.fori_loop(0, _RPT // 16, zero_body, 0)

    def zero_drain(r, carry):
        pltpu.make_async_copy(zeros_v, acc_sh.at[pl.ds(0, 16)], dsem).wait()
        return carry
    lax.fori_loop(0, _RPT // 16, zero_drain, 0)
    plsc.subcore_barrier()

    pltpu.async_copy(ones_v, acc_sh.at[dst_v.at[0]], dsem, add=True)

    def body(j, carry):
        pltpu.async_copy(ones_v, acc_sh.at[dst_v.at[j + 1]], dsem, add=True)
        pltpu.make_async_copy(ones_v, acc_sh.at[dst_v.at[0]], dsem).wait()
        return carry
    lax.fori_loop(0, _NCH - 1, body, 0)
    pltpu.make_async_copy(ones_v, acc_sh.at[dst_v.at[0]], dsem).wait()
    plsc.subcore_barrier()
    pltpu.sync_copy(acc_sh.at[pl.ds(s * _RPT, _RPT)],
                    out_hbm.at[c, pl.ds(s * _RPT, _RPT)])


def _make_spmm(d):
    @functools.partial(
        pl.kernel,
        out_type=jax.ShapeDtypeStruct((_NC, _NPAD, d), jnp.float32),
        mesh=_mesh,
        scratch_types=[
            pltpu.VMEM((_STG, _SCK), jnp.int32),
            pltpu.VMEM((_STG, _SCK), jnp.int32),
        ] + [pltpu.VMEM((_SCK, d), jnp.float32) for _ in range(_NBUF)] + [
            pltpu.VMEM((16, d), jnp.float32),
            pltpu.VMEM_SHARED((_NPAD, d), jnp.float32),
        ] + [pltpu.SemaphoreType.DMA for _ in range(_NBUF)],
    )
    def spmm(src_hbm, dst_hbm, g_hbm, out_hbm,
             src_v, dst_v, r0, r1, r2, r3, zeros_v, acc_sh, g0, g1, g2, g3):
        rows = (r0, r1, r2, r3)
        gsem = (g0, g1, g2, g3)
        c = lax.axis_index("c")
        s = lax.axis_index("s")
        for i in range(16):
            for w in range(d // 16):
                zeros_v[i, 16 * w:16 * (w + 1)] = jnp.zeros((16,), jnp.float32)

        def zero_body(r, carry):
            pltpu.async_copy(
                zeros_v, acc_sh.at[pl.ds(s * _RPT + r * 16, 16)], g0)
            return carry
        lax.fori_loop(0, _RPT // 16, zero_body, 0)

        def zero_drain(r, carry):
            pltpu.make_async_copy(zeros_v, acc_sh.at[pl.ds(0, 16)], g0).wait()
            return carry
        lax.fori_loop(0, _RPT // 16, zero_drain, 0)
        plsc.subcore_barrier()

        nsteps = lax.select(c == 0, _T0 // _STG, _T1 // _STG)
        base = lax.select(c == 0, s * _T0, _NS * _T0 + s * _T1)

        def step(st, carry):
            row0 = base + st * _STG
            pltpu.sync_copy(src_hbm.at[pl.ds(row0, _STG)], src_v)
            pltpu.sync_copy(dst_hbm.at[pl.ds(row0, _STG)], dst_v)
            for b in range(_NBUF):
                pltpu.async_copy(g_hbm.at[src_v.at[b]], rows[b], gsem[b])

            def body(i, carry2):
                for b in range(_NBUF):
                    j = _NBUF * i + b
                    pltpu.make_async_copy(
                        g_hbm.at[src_v.at[j]], rows[b], gsem[b]).wait()
                    pltpu.sync_copy(rows[b], acc_sh.at[dst_v.at[j]], add=True)

                    @pl.when(j + _NBUF < _STG)
                    def _():
                        pltpu.async_copy(
                            g_hbm.at[src_v.at[j + _NBUF]], rows[b], gsem[b])
                return carry2
            lax.fori_loop(0, _STG // _NBUF, body, 0)
            return carry
        lax.fori_loop(0, nsteps, step, 0)
        plsc.subcore_barrier()
        pltpu.sync_copy(acc_sh.at[pl.ds(s * _RPT, _RPT)],
                        out_hbm.at[c, pl.ds(s * _RPT, _RPT)])
    return spmm


_spmm_hid = _make_spmm(_D_HID)
_spmm_out = _spmm_hid


_BM = 1280
_GRID = _NPAD // _BM


def _mm_scale_body(x_ref, w_ref, degp_ref, g_ref, dinv_ref):
    deg = jnp.sum(degp_ref[...], axis=1, keepdims=True) + 1.0
    dinv = lax.rsqrt(deg)
    h = jnp.dot(x_ref[...], w_ref[...], preferred_element_type=jnp.float32)
    g_ref[...] = h * dinv
    dinv_ref[...] = dinv


def _mm_scale(x_p, w, degp):
    return pl.pallas_call(
        _mm_scale_body,
        grid=(_GRID,),
        in_specs=[
            pl.BlockSpec((_BM, _D_IN), lambda i: (i, 0)),
            pl.BlockSpec((_D_IN, _D_HID), lambda i: (0, 0)),
            pl.BlockSpec((_BM, 2), lambda i: (i, 0)),
        ],
        out_specs=[
            pl.BlockSpec((_BM, _D_HID), lambda i: (i, 0)),
            pl.BlockSpec((_BM, 1), lambda i: (i, 0)),
        ],
        out_shape=[
            jax.ShapeDtypeStruct((_NPAD, _D_HID), jnp.float32),
            jax.ShapeDtypeStruct((_NPAD, 1), jnp.float32),
        ],
    )(x_p, w, degp)


def _mid_body(acc_ref, g1_ref, dinv_ref, b1_ref, w2_ref, g2_ref):
    dinv = dinv_ref[...]
    z = (acc_ref[0] + acc_ref[1] + g1_ref[...]) * dinv + b1_ref[...]
    z = jnp.maximum(z, 0.0)
    g2_ref[...] = jnp.dot(z, w2_ref[...],
                          preferred_element_type=jnp.float32) * dinv


def _mid(acc, g1, dinv, b1, w2p):
    return pl.pallas_call(
        _mid_body,
        grid=(_GRID,),
        in_specs=[
            pl.BlockSpec((_NC, _BM, _D_HID), lambda i: (0, i, 0)),
            pl.BlockSpec((_BM, _D_HID), lambda i: (i, 0)),
            pl.BlockSpec((_BM, 1), lambda i: (i, 0)),
            pl.BlockSpec((1, _D_HID), lambda i: (0, 0)),
            pl.BlockSpec((_D_HID, _D_HID), lambda i: (0, 0)),
        ],
        out_specs=pl.BlockSpec((_BM, _D_HID), lambda i: (i, 0)),
        out_shape=jax.ShapeDtypeStruct((_NPAD, _D_HID), jnp.float32),
    )(acc, g1, dinv, b1, w2p)


def _out_body(acc_ref, g2_ref, dinv_ref, b2_ref, o_ref):
    t = (acc_ref[0] + acc_ref[1] + g2_ref[...]) * dinv_ref[...] + b2_ref[...]
    col = lax.broadcasted_iota(jnp.int32, (_BM, _D_HID), 1)
    t = jnp.where(col < _D_OUT, t, -jnp.inf)
    m = jnp.max(t, axis=1, keepdims=True)
    e = jnp.exp(t - m)
    o_ref[...] = e / jnp.sum(e, axis=1, keepdims=True)


def _out(acc, g2, dinv, b2p):
    return pl.pallas_call(
        _out_body,
        grid=(_GRID,),
        in_specs=[
            pl.BlockSpec((_NC, _BM, _D_HID), lambda i: (0, i, 0)),
            pl.BlockSpec((_BM, _D_HID), lambda i: (i, 0)),
            pl.BlockSpec((_BM, 1), lambda i: (i, 0)),
            pl.BlockSpec((1, _D_HID), lambda i: (0, 0)),
        ],
        out_specs=pl.BlockSpec((_BM, _D_HID), lambda i: (i, 0)),
        out_shape=jax.ShapeDtypeStruct((_NPAD, _D_HID), jnp.float32),
    )(acc, g2, dinv, b2p)


@jax.jit
def _run(x, edge_index, W1, b1, W2, b2):
    npad_e = _EPAD - _E
    pad_i = jnp.arange(npad_e, dtype=jnp.int32)
    src_flat = jnp.concatenate([edge_index[0], pad_i % _N])
    dst_flat = jnp.concatenate([edge_index[1], _N + pad_i % (_NPAD - _N)])
    src_p = src_flat.reshape(_NW, _NCH, _CK)
    dst_p = dst_flat.reshape(_NW, _NCH, _CK)
    src_g = src_flat.reshape(_GCH, _SCK)
    dst_g = dst_flat.reshape(_GCH, _SCK)
    x_p = jnp.concatenate(
        [x, jnp.zeros((_NPAD - _N, _D_IN), jnp.float32)], axis=0)

    deg = _deg_kernel(dst_p)
    degp = jnp.stack([deg[0, :, 0], deg[1, :, 0]], axis=1)

    w2p = jnp.concatenate(
        [W2, jnp.zeros((_D_HID, _D_HID - _D_OUT), jnp.float32)], axis=1)
    b2p = jnp.concatenate(
        [b2, jnp.zeros((_D_HID - _D_OUT,), jnp.float32)]).reshape(1, _D_HID)

    g1, dinv = _mm_scale(x_p, W1, degp)
    acc1 = _spmm_hid(src_g, dst_g, g1)
    g2 = _mid(acc1, g1, dinv, b1.reshape(1, _D_HID), w2p)
    acc2 = _spmm_out(src_g, dst_g, g2)
    out = _out(acc2, g2, dinv, b2p)
    return out[:_N, :_D_OUT]


def kernel(x, edge_index, W1, b1, W2, b2):
    return _run(x, edge_index, W1, b1, W2, b2)


# speedup vs baseline: 1.1292x; 1.1292x over previous
import functools

import jax
import jax.numpy as jnp
from jax import lax
from jax.experimental import pallas as pl
from jax.experimental.pallas import tpu as pltpu
from jax.experimental.pallas import tpu_sc as plsc

_N = 10000
_NPAD = 10240
_E = 320000
_D_IN = 128
_D_HID = 128
_D_OUT = 64

_NC = 2
_NS = 16
_NW = _NC * _NS
_CK = 128
_NCH = 80
_EPAD = _NW * _NCH * _CK
_SCK = 64
_GCH = _EPAD // _SCK
_T0 = 160
_T1 = 160
_STG = 32
_NBUF = 4
_RPT = _NPAD // _NS
_DEGW = 128

_mesh = plsc.VectorSubcoreMesh(
    core_axis_name="c", subcore_axis_name="s",
    num_cores=_NC, num_subcores=_NS)


@functools.partial(
    pl.kernel,
    out_type=jax.ShapeDtypeStruct((_NC, _NPAD, _DEGW), jnp.float32),
    mesh=_mesh,
    scratch_types=[
        pltpu.VMEM((_NCH, _CK), jnp.int32),
        pltpu.VMEM((_CK, _DEGW), jnp.float32),
        pltpu.VMEM((16, _DEGW), jnp.float32),
        pltpu.VMEM_SHARED((_NPAD, _DEGW), jnp.float32),
        pltpu.SemaphoreType.DMA,
    ],
)
def _deg_kernel(dst_hbm, out_hbm, dst_v, ones_v, zeros_v, acc_sh, dsem):
    c = lax.axis_index("c")
    s = lax.axis_index("s")
    tid = c * _NS + s
    pltpu.sync_copy(dst_hbm.at[tid], dst_v)
    for i in range(_CK):
        for w in range(_DEGW // 16):
            ones_v[i, 16 * w:16 * (w + 1)] = jnp.ones((16,), jnp.float32)
    for i in range(16):
        for w in range(_DEGW // 16):
            zeros_v[i, 16 * w:16 * (w + 1)] = jnp.zeros((16,), jnp.float32)

    def zero_body(r, carry):
        pltpu.async_copy(zeros_v, acc_sh.at[pl.ds(s * _RPT + r * 16, 16)], dsem)
        return carry
    lax.fori_loop(0, _RPT // 16, zero_body, 0)

    def zero_drain(r, carry):
        pltpu.make_async_copy(zeros_v, acc_sh.at[pl.ds(0, 16)], dsem).wait()
        return carry
    lax.fori_loop(0, _RPT // 16, zero_drain, 0)
    plsc.subcore_barrier()

    pltpu.async_copy(ones_v, acc_sh.at[dst_v.at[0]], dsem, add=True)

    def body(j, carry):
        pltpu.async_copy(ones_v, acc_sh.at[dst_v.at[j + 1]], dsem, add=True)
        pltpu.make_async_copy(ones_v, acc_sh.at[dst_v.at[0]], dsem).wait()
        return carry
    lax.fori_loop(0, _NCH - 1, body, 0)
    pltpu.make_async_copy(ones_v, acc_sh.at[dst_v.at[0]], dsem).wait()
    plsc.subcore_barrier()
    pltpu.sync_copy(acc_sh.at[pl.ds(s * _RPT, _RPT)],
                    out_hbm.at[c, pl.ds(s * _RPT, _RPT)])


def _make_spmm(d):
    @functools.partial(
        pl.kernel,
        out_type=jax.ShapeDtypeStruct((_NC, _NPAD, d), jnp.float32),
        mesh=_mesh,
        scratch_types=[
            pltpu.VMEM((_STG, _SCK), jnp.int32),
            pltpu.VMEM((_STG, _SCK), jnp.int32),
        ] + [pltpu.VMEM((_SCK, d), jnp.float32) for _ in range(_NBUF)] + [
            pltpu.VMEM((16, d), jnp.float32),
            pltpu.VMEM_SHARED((_NPAD, d), jnp.float32),
        ] + [pltpu.SemaphoreType.DMA for _ in range(_NBUF)],
    )
    def spmm(src_hbm, dst_hbm, g_hbm, out_hbm,
             src_v, dst_v, r0, r1, r2, r3, zeros_v, acc_sh, g0, g1, g2, g3):
        rows = (r0, r1, r2, r3)
        gsem = (g0, g1, g2, g3)
        c = lax.axis_index("c")
        s = lax.axis_index("s")
        for i in range(16):
            for w in range(d // 16):
                zeros_v[i, 16 * w:16 * (w + 1)] = jnp.zeros((16,), jnp.float32)

        def zero_body(r, carry):
            pltpu.async_copy(
                zeros_v, acc_sh.at[pl.ds(s * _RPT + r * 16, 16)], g0)
            return carry
        lax.fori_loop(0, _RPT // 16, zero_body, 0)

        def zero_drain(r, carry):
            pltpu.make_async_copy(zeros_v, acc_sh.at[pl.ds(0, 16)], g0).wait()
            return carry
        lax.fori_loop(0, _RPT // 16, zero_drain, 0)
        plsc.subcore_barrier()

        nsteps = lax.select(c == 0, _T0 // _STG, _T1 // _STG)
        base = lax.select(c == 0, s * _T0, _NS * _T0 + s * _T1)

        def step(st, carry):
            row0 = base + st * _STG
            pltpu.sync_copy(src_hbm.at[pl.ds(row0, _STG)], src_v)
            pltpu.sync_copy(dst_hbm.at[pl.ds(row0, _STG)], dst_v)
            for b in range(_NBUF):
                pltpu.async_copy(g_hbm.at[src_v.at[b]], rows[b], gsem[b])

            def body(i, carry2):
                for b in range(_NBUF):
                    j = _NBUF * i + b
                    pltpu.make_async_copy(
                        g_hbm.at[src_v.at[j]], rows[b], gsem[b]).wait()
                    pltpu.sync_copy(rows[b], acc_sh.at[dst_v.at[j]], add=True)

                    @pl.when(j + _NBUF < _STG)
                    def _():
                        pltpu.async_copy(
                            g_hbm.at[src_v.at[j + _NBUF]], rows[b], gsem[b])
                return carry2
            lax.fori_loop(0, _STG // _NBUF, body, 0)
            return carry
        lax.fori_loop(0, nsteps, step, 0)
        plsc.subcore_barrier()
        pltpu.sync_copy(acc_sh.at[pl.ds(s * _RPT, _RPT)],
                        out_hbm.at[c, pl.ds(s * _RPT, _RPT)])
    return spmm


_spmm_hid = _make_spmm(_D_HID)
_spmm_out = _spmm_hid


_BM = 1280
_GRID = _NPAD // _BM


def _mm_scale_body(x_ref, w_ref, degp_ref, g_ref, dinv_ref):
    deg = jnp.sum(degp_ref[...], axis=1, keepdims=True) + 1.0
    dinv = lax.rsqrt(deg)
    h = jnp.dot(x_ref[...], w_ref[...], preferred_element_type=jnp.float32)
    g_ref[...] = h * dinv
    dinv_ref[...] = dinv


def _mm_scale(x_p, w, degp):
    return pl.pallas_call(
        _mm_scale_body,
        grid=(_GRID,),
        in_specs=[
            pl.BlockSpec((_BM, _D_IN), lambda i: (i, 0)),
            pl.BlockSpec((_D_IN, _D_HID), lambda i: (0, 0)),
            pl.BlockSpec((_BM, 2), lambda i: (i, 0)),
        ],
        out_specs=[
            pl.BlockSpec((_BM, _D_HID), lambda i: (i, 0)),
            pl.BlockSpec((_BM, 1), lambda i: (i, 0)),
        ],
        out_shape=[
            jax.ShapeDtypeStruct((_NPAD, _D_HID), jnp.float32),
            jax.ShapeDtypeStruct((_NPAD, 1), jnp.float32),
        ],
    )(x_p, w, degp)


def _mid_body(acc_ref, g1_ref, dinv_ref, b1_ref, w2_ref, g2_ref):
    dinv = dinv_ref[...]
    z = (acc_ref[0] + acc_ref[1] + g1_ref[...]) * dinv + b1_ref[...]
    z = jnp.maximum(z, 0.0)
    g2_ref[...] = jnp.dot(z, w2_ref[...],
                          preferred_element_type=jnp.float32) * dinv


def _mid(acc, g1, dinv, b1, w2p):
    return pl.pallas_call(
        _mid_body,
        grid=(_GRID,),
        in_specs=[
            pl.BlockSpec((_NC, _BM, _D_HID), lambda i: (0, i, 0)),
            pl.BlockSpec((_BM, _D_HID), lambda i: (i, 0)),
            pl.BlockSpec((_BM, 1), lambda i: (i, 0)),
            pl.BlockSpec((1, _D_HID), lambda i: (0, 0)),
            pl.BlockSpec((_D_HID, _D_HID), lambda i: (0, 0)),
        ],
        out_specs=pl.BlockSpec((_BM, _D_HID), lambda i: (i, 0)),
        out_shape=jax.ShapeDtypeStruct((_NPAD, _D_HID), jnp.float32),
    )(acc, g1, dinv, b1, w2p)


def _out_body(acc_ref, g2_ref, dinv_ref, b2_ref, o_ref):
    t = (acc_ref[0] + acc_ref[1] + g2_ref[...]) * dinv_ref[...] + b2_ref[...]
    col = lax.broadcasted_iota(jnp.int32, (_BM, _D_HID), 1)
    t = jnp.where(col < _D_OUT, t, -jnp.inf)
    m = jnp.max(t, axis=1, keepdims=True)
    e = jnp.exp(t - m)
    o_ref[...] = e / jnp.sum(e, axis=1, keepdims=True)


def _out(acc, g2, dinv, b2p):
    return pl.pallas_call(
        _out_body,
        grid=(_GRID,),
        in_specs=[
            pl.BlockSpec((_NC, _BM, _D_HID), lambda i: (0, i, 0)),
            pl.BlockSpec((_BM, _D_HID), lambda i: (i, 0)),
            pl.BlockSpec((_BM, 1), lambda i: (i, 0)),
            pl.BlockSpec((1, _D_HID), lambda i: (0, 0)),
        ],
        out_specs=pl.BlockSpec((_BM, _D_HID), lambda i: (i, 0)),
        out_shape=jax.ShapeDtypeStruct((_NPAD, _D_HID), jnp.float32),
    )(acc, g2, dinv, b2p)


@jax.jit
def _run(x, edge_index, W1, b1, W2, b2):
    npad_e = _EPAD - _E
    pad_i = jnp.arange(npad_e, dtype=jnp.int32)
    src_flat = jnp.concatenate([edge_index[0], pad_i % _N])
    dst_flat = jnp.concatenate([edge_index[1], _N + pad_i % (_NPAD - _N)])
    src_p = src_flat.reshape(_NW, _NCH, _CK)
    dst_p = dst_flat.reshape(_NW, _NCH, _CK)
    src_g = src_flat.reshape(_GCH, _SCK)
    dst_g = dst_flat.reshape(_GCH, _SCK)
    x_p = jnp.concatenate(
        [x, jnp.zeros((_NPAD - _N, _D_IN), jnp.float32)], axis=0)

    deg = _deg_kernel(dst_p)
    degp = jnp.stack([deg[0, :, 0], deg[1, :, 0]], axis=1)

    w2p = jnp.concatenate(
        [W2, jnp.zeros((_D_HID, _D_HID - _D_OUT), jnp.float32)], axis=1)
    b2p = jnp.concatenate(
        [b2, jnp.zeros((_D_HID - _D_OUT,), jnp.float32)]).reshape(1, _D_HID)

    g1, dinv = _mm_scale(x_p, W1, degp)
    acc1 = _spmm_hid(src_g, dst_g, g1)
    g2 = _mid(acc1, g1, dinv, b1.reshape(1, _D_HID), w2p)
    acc2 = _spmm_out(src_g, dst_g, g2)
    out = _out(acc2, g2, dinv, b2p)
    return out[:_N, :_D_OUT]


def kernel(x, edge_index, W1, b1, W2, b2):
    return _run(x, edge_index, W1, b1, W2, b2)
